# baseline (speedup 1.0000x reference)
import functools

import jax
import jax.numpy as jnp
from jax import lax
from jax.experimental import pallas as pl
from jax.experimental.pallas import tpu as pltpu
from jax.experimental.pallas import tpu_sc as plsc

USER_N = 50000
ITEM_N = 50000
NODE_N = USER_N + ITEM_N
DIM = 32
HALF = 16
N_EDGES = 1600000
N_LAYERS = 3

NC = 2
NS = 16
LANES = 16

SUB = 128
CHUNK_ROWS = 4
CHUNK = SUB * CHUNK_ROWS
N_CHUNKS = 200
EDGES_PER_TILE = N_CHUNKS * CHUNK
EDGES_PAD = EDGES_PER_TILE * NS
ROWS_PER_TILE = EDGES_PER_TILE // SUB
NPT = 6256
NPT_LAST = NODE_N - NPT * (NS - 1)


_GDN = lax.GatherDimensionNumbers(
    offset_dims=(), collapsed_slice_dims=(0,), start_index_map=(0,))


def _splat(v, i):
    idx = jnp.full((LANES, 1), i, jnp.int32)
    return lax.gather(v, idx, _GDN, (1,),
                      mode=lax.GatherScatterMode.PROMISE_IN_BOUNDS)


def _sc_body(emb_in, col2, row2, w2, zeros_hbm, out1, out2, out3,
             acc, colbuf0, colbuf1, rowbuf0, rowbuf1, wbuf0, wbuf1,
             msg0, msg1, sem_s0, sem_s1, sem_g0, sem_g1, sem_a0, sem_a1):
    c_idx = lax.axis_index("c")
    s_idx = lax.axis_index("s")
    core_off = c_idx * NODE_N
    erow0 = s_idx * ROWS_PER_TILE
    nbase = s_idx * NPT
    is_last = s_idx == NS - 1

    colbufs = (colbuf0, colbuf1)
    rowbufs = (rowbuf0, rowbuf1)
    wbufs = (wbuf0, wbuf1)
    msgs = (msg0, msg1)
    sem_s = (sem_s0, sem_s1)
    sem_g = (sem_g0, sem_g1)
    sem_a = (sem_a0, sem_a1)

    def stage_descs(n, b):
        r0 = erow0 + n * CHUNK_ROWS
        sl = pl.ds(r0, CHUNK_ROWS)
        return ((col2.at[sl], colbufs[b]), (row2.at[sl], rowbufs[b]),
                (w2.at[sl], wbufs[b]))

    def fire_stage(n, b):
        for src, dst in stage_descs(n, b):
            pltpu.async_copy(src, dst, sem_s[b])

    def wait_stage(n, b):
        for src, dst in stage_descs(n, b):
            pltpu.make_async_copy(src, dst, sem_s[b]).wait()

    def gather_descs(b, src_view):
        return [(src_view.at[colbufs[b].at[j]], msgs[b].at[pl.ds(j * SUB, SUB)])
                for j in range(CHUNK_ROWS)]

    def fire_gather(b, src_view):
        for src, dst in gather_descs(b, src_view):
            pltpu.async_copy(src, dst, sem_g[b])

    def wait_gather(b, src_view):
        for src, dst in gather_descs(b, src_view):
            pltpu.make_async_copy(src, dst, sem_g[b]).wait()

    def scatter_descs(b):
        return [(msgs[b].at[pl.ds(j * SUB, SUB)], acc.at[rowbufs[b].at[j]])
                for j in range(CHUNK_ROWS)]

    def fire_scatter(b):
        for src, dst in scatter_descs(b):
            pltpu.async_copy(src, dst, sem_a[b], add=True)

    def wait_scatter(b):
        for src, dst in scatter_descs(b):
            pltpu.make_async_copy(src, dst, sem_a[b]).wait()

    def compute(b):
        msg, wbuf = msgs[b], wbufs[b]

        def group(g, carry):
            j = lax.shift_right_logical(g, 3)
            k16 = lax.bitwise_and(g, 7) * LANES
            wv = wbuf[j, pl.ds(k16, LANES)]
            base = g * LANES
            for i in range(LANES):
                msg[base + i, :] = msg[base + i, :] * _splat(wv, i)
            return carry
        lax.fori_loop(0, CHUNK // LANES, group, 0)

    def run_layer(src_ref, dst_ref):
        @pl.when(jnp.logical_not(is_last))
        def _():
            pltpu.sync_copy(zeros_hbm, acc.at[pl.ds(nbase, NPT)])

        @pl.when(is_last)
        def _():
            pltpu.sync_copy(zeros_hbm.at[pl.ds(0, NPT_LAST)],
                            acc.at[pl.ds(nbase, NPT_LAST)])
        plsc.subcore_barrier()

        src_view = src_ref.at[pl.ds(core_off, NODE_N)]

        def step(n, b, first=False, last=False):
            if not first:
                wait_scatter(b ^ 1)
            if not last:
                fire_stage(n + 1, b ^ 1)
            wait_gather(b, src_view)
            if not last:
                wait_stage(n + 1, b ^ 1)
                fire_gather(b ^ 1, src_view)
            compute(b)
            fire_scatter(b)

        fire_stage(0, 0)
        wait_stage(0, 0)
        fire_gather(0, src_view)
        step(0, 0, first=True)

        def pair(t, carry):
            step(2 * t + 1, 1)
            step(2 * t + 2, 0)
            return carry
        lax.fori_loop(0, (N_CHUNKS - 2) // 2, pair, 0)

        step(N_CHUNKS - 1, 1, last=True)
        wait_scatter(1)

        plsc.subcore_barrier()

        @pl.when(jnp.logical_not(is_last))
        def _():
            pltpu.sync_copy(acc.at[pl.ds(nbase, NPT)],
                            dst_ref.at[pl.ds(core_off + nbase, NPT)])

        @pl.when(is_last)
        def _():
            pltpu.sync_copy(acc.at[pl.ds(nbase, NPT_LAST)],
                            dst_ref.at[pl.ds(core_off + nbase, NPT_LAST)])

    run_layer(emb_in, out1)
    plsc.subcore_barrier()
    run_layer(out1, out2)
    plsc.subcore_barrier()
    run_layer(out2, out3)


_emb_t = jax.ShapeDtypeStruct((NC * NODE_N, HALF), jnp.float32)

_sc_kernel = pl.kernel(
    _sc_body,
    out_type=(_emb_t, _emb_t, _emb_t),
    mesh=plsc.VectorSubcoreMesh(core_axis_name="c", subcore_axis_name="s",
                                num_cores=NC, num_subcores=NS),
    compiler_params=pltpu.CompilerParams(use_tc_tiling_on_sc=False),
    scratch_types=(
        [pltpu.VMEM_SHARED((NODE_N, HALF), jnp.float32)]
        + [pltpu.VMEM((CHUNK_ROWS, SUB), jnp.int32)] * 4
        + [pltpu.VMEM((CHUNK_ROWS, SUB), jnp.float32)] * 2
        + [pltpu.VMEM((CHUNK, HALF), jnp.float32)] * 2
        + [pltpu.SemaphoreType.DMA] * 6
    ),
)


_MB = 125


def _mean_body(a0, b0, c0, d0, a1, b1, c1, d1, o_ref):
    lo = (a0[...] + b0[...] + c0[...] + d0[...]) * 0.25
    hi = (a1[...] + b1[...] + c1[...] + d1[...]) * 0.25
    lo3 = lo.reshape(_MB, 8, HALF)
    hi3 = hi.reshape(_MB, 8, HALF)
    o_ref[...] = jnp.concatenate([lo3, hi3], axis=2)[None]


def _layer_mean(e0, l1, l2, l3):
    nb = (NODE_N * HALF) // (128 * _MB)
    args = [x.reshape(2, nb, _MB, 128) for x in (e0, l1, l2, l3)]
    blk_lo = pl.BlockSpec((1, 1, _MB, 128), lambda i: (0, i, 0, 0))
    blk_hi = pl.BlockSpec((1, 1, _MB, 128), lambda i: (1, i, 0, 0))
    out = pl.pallas_call(
        _mean_body,
        grid=(nb,),
        in_specs=[blk_lo] * 4 + [blk_hi] * 4,
        out_specs=pl.BlockSpec((1, _MB, 8, DIM), lambda i: (i, 0, 0, 0)),
        out_shape=jax.ShapeDtypeStruct((nb, _MB, 8, DIM), jnp.float32),
    )(*args, *args)
    return out.reshape(NODE_N, DIM)


def kernel(embed_user, embed_item, edge_weight, edge_index):
    emb_in = jnp.concatenate(
        [embed_user[:, :HALF], embed_item[:, :HALF],
         embed_user[:, HALF:], embed_item[:, HALF:]], axis=0)

    pad = EDGES_PAD - N_EDGES
    col2 = jnp.pad(edge_index[1], (0, pad)).reshape(EDGES_PAD // SUB, SUB)
    row2 = jnp.pad(edge_index[0], (0, pad)).reshape(EDGES_PAD // SUB, SUB)
    w2 = jnp.pad(edge_weight, (0, pad)).reshape(EDGES_PAD // SUB, SUB)
    zeros_hbm = jnp.zeros((NPT, HALF), jnp.float32)

    l1, l2, l3 = _sc_kernel(emb_in, col2, row2, w2, zeros_hbm)
    full = _layer_mean(emb_in, l1, l2, l3)
    return full[:USER_N], full[USER_N:]


# speedup vs baseline: 1.0002x; 1.0002x over previous
import functools

import jax
import jax.numpy as jnp
from jax import lax
from jax.experimental import pallas as pl
from jax.experimental.pallas import tpu as pltpu
from jax.experimental.pallas import tpu_sc as plsc

USER_N = 50000
ITEM_N = 50000
NODE_N = USER_N + ITEM_N
DIM = 32
HALF = 16
N_EDGES = 1600000
N_LAYERS = 3

NC = 2
NS = 16
LANES = 16

SUB = 128
CHUNK_ROWS = 4
CHUNK = SUB * CHUNK_ROWS
N_CHUNKS = 200
EDGES_PER_TILE = N_CHUNKS * CHUNK
EDGES_PAD = EDGES_PER_TILE * NS
ROWS_PER_TILE = EDGES_PER_TILE // SUB
NPT = 6256
NPT_LAST = NODE_N - NPT * (NS - 1)


_GDN = lax.GatherDimensionNumbers(
    offset_dims=(), collapsed_slice_dims=(0,), start_index_map=(0,))


def _splat(v, i):
    idx = jnp.full((LANES, 1), i, jnp.int32)
    return lax.gather(v, idx, _GDN, (1,),
                      mode=lax.GatherScatterMode.PROMISE_IN_BOUNDS)


def _sc_body(emb_in, col2, row2, w2, zeros_hbm, out1, out2, out3,
             acc, colbuf0, colbuf1, rowbuf0, rowbuf1, wbuf0, wbuf1,
             msg0, msg1, sem_s0, sem_s1, sem_g0, sem_g1, sem_a0, sem_a1):
    c_idx = lax.axis_index("c")
    s_idx = lax.axis_index("s")
    core_off = c_idx * NODE_N
    erow0 = s_idx * ROWS_PER_TILE
    nbase = s_idx * NPT
    is_last = s_idx == NS - 1

    colbufs = (colbuf0, colbuf1)
    rowbufs = (rowbuf0, rowbuf1)
    wbufs = (wbuf0, wbuf1)
    msgs = (msg0, msg1)
    sem_s = (sem_s0, sem_s1)
    sem_g = (sem_g0, sem_g1)
    sem_a = (sem_a0, sem_a1)

    def stage_descs(n, b):
        r0 = erow0 + n * CHUNK_ROWS
        sl = pl.ds(r0, CHUNK_ROWS)
        return ((col2.at[sl], colbufs[b]), (row2.at[sl], rowbufs[b]),
                (w2.at[sl], wbufs[b]))

    def fire_stage(n, b):
        for src, dst in stage_descs(n, b):
            pltpu.async_copy(src, dst, sem_s[b])

    def wait_stage(n, b):
        for src, dst in stage_descs(n, b):
            pltpu.make_async_copy(src, dst, sem_s[b]).wait()

    def gather_descs(b, src_view):
        return [(src_view.at[colbufs[b].at[j]], msgs[b].at[pl.ds(j * SUB, SUB)])
                for j in range(CHUNK_ROWS)]

    def fire_gather(b, src_view):
        for src, dst in gather_descs(b, src_view):
            pltpu.async_copy(src, dst, sem_g[b])

    def wait_gather(b, src_view):
        for src, dst in gather_descs(b, src_view):
            pltpu.make_async_copy(src, dst, sem_g[b]).wait()

    def scatter_descs(b):
        return [(msgs[b].at[pl.ds(j * SUB, SUB)], acc.at[rowbufs[b].at[j]])
                for j in range(CHUNK_ROWS)]

    def fire_scatter(b):
        for src, dst in scatter_descs(b):
            pltpu.async_copy(src, dst, sem_a[b], add=True)

    def wait_scatter(b):
        for src, dst in scatter_descs(b):
            pltpu.make_async_copy(src, dst, sem_a[b]).wait()

    def compute(b):
        msg, wbuf = msgs[b], wbufs[b]

        def group(g, carry):
            j = lax.shift_right_logical(g, 3)
            k16 = lax.bitwise_and(g, 7) * LANES
            wv = wbuf[j, pl.ds(k16, LANES)]
            base = g * LANES
            for i in range(LANES):
                msg[base + i, :] = msg[base + i, :] * wv[i]
            return carry
        lax.fori_loop(0, CHUNK // LANES, group, 0)

    def run_layer(src_ref, dst_ref):
        @pl.when(jnp.logical_not(is_last))
        def _():
            pltpu.sync_copy(zeros_hbm, acc.at[pl.ds(nbase, NPT)])

        @pl.when(is_last)
        def _():
            pltpu.sync_copy(zeros_hbm.at[pl.ds(0, NPT_LAST)],
                            acc.at[pl.ds(nbase, NPT_LAST)])
        plsc.subcore_barrier()

        src_view = src_ref.at[pl.ds(core_off, NODE_N)]

        def step(n, b, first=False, last=False):
            if not first:
                wait_scatter(b ^ 1)
            if not last:
                fire_stage(n + 1, b ^ 1)
            wait_gather(b, src_view)
            if not last:
                wait_stage(n + 1, b ^ 1)
                fire_gather(b ^ 1, src_view)
            compute(b)
            fire_scatter(b)

        fire_stage(0, 0)
        wait_stage(0, 0)
        fire_gather(0, src_view)
        step(0, 0, first=True)

        def pair(t, carry):
            step(2 * t + 1, 1)
            step(2 * t + 2, 0)
            return carry
        lax.fori_loop(0, (N_CHUNKS - 2) // 2, pair, 0)

        step(N_CHUNKS - 1, 1, last=True)
        wait_scatter(1)

        plsc.subcore_barrier()

        @pl.when(jnp.logical_not(is_last))
        def _():
            pltpu.sync_copy(acc.at[pl.ds(nbase, NPT)],
                            dst_ref.at[pl.ds(core_off + nbase, NPT)])

        @pl.when(is_last)
        def _():
            pltpu.sync_copy(acc.at[pl.ds(nbase, NPT_LAST)],
                            dst_ref.at[pl.ds(core_off + nbase, NPT_LAST)])

    run_layer(emb_in, out1)
    plsc.subcore_barrier()
    run_layer(out1, out2)
    plsc.subcore_barrier()
    run_layer(out2, out3)


_emb_t = jax.ShapeDtypeStruct((NC * NODE_N, HALF), jnp.float32)

_sc_kernel = pl.kernel(
    _sc_body,
    out_type=(_emb_t, _emb_t, _emb_t),
    mesh=plsc.VectorSubcoreMesh(core_axis_name="c", subcore_axis_name="s",
                                num_cores=NC, num_subcores=NS),
    compiler_params=pltpu.CompilerParams(use_tc_tiling_on_sc=False),
    scratch_types=(
        [pltpu.VMEM_SHARED((NODE_N, HALF), jnp.float32)]
        + [pltpu.VMEM((CHUNK_ROWS, SUB), jnp.int32)] * 4
        + [pltpu.VMEM((CHUNK_ROWS, SUB), jnp.float32)] * 2
        + [pltpu.VMEM((CHUNK, HALF), jnp.float32)] * 2
        + [pltpu.SemaphoreType.DMA] * 6
    ),
)


_MB = 125


def _mean_body(a0, b0, c0, d0, a1, b1, c1, d1, o_ref):
    lo = (a0[...] + b0[...] + c0[...] + d0[...]) * 0.25
    hi = (a1[...] + b1[...] + c1[...] + d1[...]) * 0.25
    lo3 = lo.reshape(_MB, 8, HALF)
    hi3 = hi.reshape(_MB, 8, HALF)
    o_ref[...] = jnp.concatenate([lo3, hi3], axis=2)[None]


def _layer_mean(e0, l1, l2, l3):
    nb = (NODE_N * HALF) // (128 * _MB)
    args = [x.reshape(2, nb, _MB, 128) for x in (e0, l1, l2, l3)]
    blk_lo = pl.BlockSpec((1, 1, _MB, 128), lambda i: (0, i, 0, 0))
    blk_hi = pl.BlockSpec((1, 1, _MB, 128), lambda i: (1, i, 0, 0))
    out = pl.pallas_call(
        _mean_body,
        grid=(nb,),
        in_specs=[blk_lo] * 4 + [blk_hi] * 4,
        out_specs=pl.BlockSpec((1, _MB, 8, DIM), lambda i: (i, 0, 0, 0)),
        out_shape=jax.ShapeDtypeStruct((nb, _MB, 8, DIM), jnp.float32),
    )(*args, *args)
    return out.reshape(NODE_N, DIM)


def kernel(embed_user, embed_item, edge_weight, edge_index):
    emb_in = jnp.concatenate(
        [embed_user[:, :HALF], embed_item[:, :HALF],
         embed_user[:, HALF:], embed_item[:, HALF:]], axis=0)

    pad = EDGES_PAD - N_EDGES
    col2 = jnp.pad(edge_index[1], (0, pad)).reshape(EDGES_PAD // SUB, SUB)
    row2 = jnp.pad(edge_index[0], (0, pad)).reshape(EDGES_PAD // SUB, SUB)
    w2 = jnp.pad(edge_weight, (0, pad)).reshape(EDGES_PAD // SUB, SUB)
    zeros_hbm = jnp.zeros((NPT, HALF), jnp.float32)

    l1, l2, l3 = _sc_kernel(emb_in, col2, row2, w2, zeros_hbm)
    full = _layer_mean(emb_in, l1, l2, l3)
    return full[:USER_N], full[USER_N:]


# speedup vs baseline: 1.0304x; 1.0302x over previous
import functools

import jax
import jax.numpy as jnp
from jax import lax
from jax.experimental import pallas as pl
from jax.experimental.pallas import tpu as pltpu
from jax.experimental.pallas import tpu_sc as plsc

USER_N = 50000
ITEM_N = 50000
NODE_N = USER_N + ITEM_N
DIM = 32
HALF = 16
N_EDGES = 1600000
N_LAYERS = 3

NC = 2
NS = 16
LANES = 16

SUB = 128
CHUNK_ROWS = 5
CHUNK = SUB * CHUNK_ROWS
N_CHUNKS = 160
EDGES_PER_TILE = N_CHUNKS * CHUNK
EDGES_PAD = EDGES_PER_TILE * NS
ROWS_PER_TILE = EDGES_PER_TILE // SUB
NPT = 6256
NPT_LAST = NODE_N - NPT * (NS - 1)


_GDN = lax.GatherDimensionNumbers(
    offset_dims=(), collapsed_slice_dims=(0,), start_index_map=(0,))


def _splat(v, i):
    idx = jnp.full((LANES, 1), i, jnp.int32)
    return lax.gather(v, idx, _GDN, (1,),
                      mode=lax.GatherScatterMode.PROMISE_IN_BOUNDS)


def _sc_body(emb_in, col2, row2, w2, zeros_hbm, out1, out2, out3,
             acc, colbuf0, colbuf1, rowbuf0, rowbuf1, wbuf0, wbuf1,
             msg0, msg1, sem_s0, sem_s1, sem_g0, sem_g1, sem_a0, sem_a1):
    c_idx = lax.axis_index("c")
    s_idx = lax.axis_index("s")
    core_off = c_idx * NODE_N
    erow0 = s_idx * ROWS_PER_TILE
    nbase = s_idx * NPT
    is_last = s_idx == NS - 1

    colbufs = (colbuf0, colbuf1)
    rowbufs = (rowbuf0, rowbuf1)
    wbufs = (wbuf0, wbuf1)
    msgs = (msg0, msg1)
    sem_s = (sem_s0, sem_s1)
    sem_g = (sem_g0, sem_g1)
    sem_a = (sem_a0, sem_a1)

    def stage_descs(n, b):
        r0 = erow0 + n * CHUNK_ROWS
        sl = pl.ds(r0, CHUNK_ROWS)
        return ((col2.at[sl], colbufs[b]), (row2.at[sl], rowbufs[b]),
                (w2.at[sl], wbufs[b]))

    def fire_stage(n, b):
        for src, dst in stage_descs(n, b):
            pltpu.async_copy(src, dst, sem_s[b])

    def wait_stage(n, b):
        for src, dst in stage_descs(n, b):
            pltpu.make_async_copy(src, dst, sem_s[b]).wait()

    def gather_descs(b, src_view):
        return [(src_view.at[colbufs[b].at[j]], msgs[b].at[pl.ds(j * SUB, SUB)])
                for j in range(CHUNK_ROWS)]

    def fire_gather(b, src_view):
        for src, dst in gather_descs(b, src_view):
            pltpu.async_copy(src, dst, sem_g[b])

    def wait_gather(b, src_view):
        for src, dst in gather_descs(b, src_view):
            pltpu.make_async_copy(src, dst, sem_g[b]).wait()

    def scatter_descs(b):
        return [(msgs[b].at[pl.ds(j * SUB, SUB)], acc.at[rowbufs[b].at[j]])
                for j in range(CHUNK_ROWS)]

    def fire_scatter(b):
        for src, dst in scatter_descs(b):
            pltpu.async_copy(src, dst, sem_a[b], add=True)

    def wait_scatter(b):
        for src, dst in scatter_descs(b):
            pltpu.make_async_copy(src, dst, sem_a[b]).wait()

    def compute(b):
        msg, wbuf = msgs[b], wbufs[b]

        def group(g, carry):
            j = lax.shift_right_logical(g, 3)
            k16 = lax.bitwise_and(g, 7) * LANES
            wv = wbuf[j, pl.ds(k16, LANES)]
            base = g * LANES
            for i in range(LANES):
                msg[base + i, :] = msg[base + i, :] * wv[i]
            return carry
        lax.fori_loop(0, CHUNK // LANES, group, 0)

    def run_layer(src_ref, dst_ref):
        @pl.when(jnp.logical_not(is_last))
        def _():
            pltpu.sync_copy(zeros_hbm, acc.at[pl.ds(nbase, NPT)])

        @pl.when(is_last)
        def _():
            pltpu.sync_copy(zeros_hbm.at[pl.ds(0, NPT_LAST)],
                            acc.at[pl.ds(nbase, NPT_LAST)])
        plsc.subcore_barrier()

        src_view = src_ref.at[pl.ds(core_off, NODE_N)]

        def step(n, b, first=False, last=False):
            if not first:
                wait_scatter(b ^ 1)
            if not last:
                fire_stage(n + 1, b ^ 1)
            wait_gather(b, src_view)
            if not last:
                wait_stage(n + 1, b ^ 1)
                fire_gather(b ^ 1, src_view)
            compute(b)
            fire_scatter(b)

        fire_stage(0, 0)
        wait_stage(0, 0)
        fire_gather(0, src_view)
        step(0, 0, first=True)

        def pair(t, carry):
            step(2 * t + 1, 1)
            step(2 * t + 2, 0)
            return carry
        lax.fori_loop(0, (N_CHUNKS - 2) // 2, pair, 0)

        step(N_CHUNKS - 1, 1, last=True)
        wait_scatter(1)

        plsc.subcore_barrier()

        @pl.when(jnp.logical_not(is_last))
        def _():
            pltpu.sync_copy(acc.at[pl.ds(nbase, NPT)],
                            dst_ref.at[pl.ds(core_off + nbase, NPT)])

        @pl.when(is_last)
        def _():
            pltpu.sync_copy(acc.at[pl.ds(nbase, NPT_LAST)],
                            dst_ref.at[pl.ds(core_off + nbase, NPT_LAST)])

    run_layer(emb_in, out1)
    plsc.subcore_barrier()
    run_layer(out1, out2)
    plsc.subcore_barrier()
    run_layer(out2, out3)


_emb_t = jax.ShapeDtypeStruct((NC * NODE_N, HALF), jnp.float32)

_sc_kernel = pl.kernel(
    _sc_body,
    out_type=(_emb_t, _emb_t, _emb_t),
    mesh=plsc.VectorSubcoreMesh(core_axis_name="c", subcore_axis_name="s",
                                num_cores=NC, num_subcores=NS),
    compiler_params=pltpu.CompilerParams(use_tc_tiling_on_sc=False),
    scratch_types=(
        [pltpu.VMEM_SHARED((NODE_N, HALF), jnp.float32)]
        + [pltpu.VMEM((CHUNK_ROWS, SUB), jnp.int32)] * 4
        + [pltpu.VMEM((CHUNK_ROWS, SUB), jnp.float32)] * 2
        + [pltpu.VMEM((CHUNK, HALF), jnp.float32)] * 2
        + [pltpu.SemaphoreType.DMA] * 6
    ),
)


_MB = 125


def _mean_body(a0, b0, c0, d0, a1, b1, c1, d1, o_ref):
    lo = (a0[...] + b0[...] + c0[...] + d0[...]) * 0.25
    hi = (a1[...] + b1[...] + c1[...] + d1[...]) * 0.25
    lo3 = lo.reshape(_MB, 8, HALF)
    hi3 = hi.reshape(_MB, 8, HALF)
    o_ref[...] = jnp.concatenate([lo3, hi3], axis=2)[None]


def _layer_mean(e0, l1, l2, l3):
    nb = (NODE_N * HALF) // (128 * _MB)
    args = [x.reshape(2, nb, _MB, 128) for x in (e0, l1, l2, l3)]
    blk_lo = pl.BlockSpec((1, 1, _MB, 128), lambda i: (0, i, 0, 0))
    blk_hi = pl.BlockSpec((1, 1, _MB, 128), lambda i: (1, i, 0, 0))
    out = pl.pallas_call(
        _mean_body,
        grid=(nb,),
        in_specs=[blk_lo] * 4 + [blk_hi] * 4,
        out_specs=pl.BlockSpec((1, _MB, 8, DIM), lambda i: (i, 0, 0, 0)),
        out_shape=jax.ShapeDtypeStruct((nb, _MB, 8, DIM), jnp.float32),
    )(*args, *args)
    return out.reshape(NODE_N, DIM)


def kernel(embed_user, embed_item, edge_weight, edge_index):
    emb_in = jnp.concatenate(
        [embed_user[:, :HALF], embed_item[:, :HALF],
         embed_user[:, HALF:], embed_item[:, HALF:]], axis=0)

    pad = EDGES_PAD - N_EDGES
    col2 = jnp.pad(edge_index[1], (0, pad)).reshape(EDGES_PAD // SUB, SUB)
    row2 = jnp.pad(edge_index[0], (0, pad)).reshape(EDGES_PAD // SUB, SUB)
    w2 = jnp.pad(edge_weight, (0, pad)).reshape(EDGES_PAD // SUB, SUB)
    zeros_hbm = jnp.zeros((NPT, HALF), jnp.float32)

    l1, l2, l3 = _sc_kernel(emb_in, col2, row2, w2, zeros_hbm)
    full = _layer_mean(emb_in, l1, l2, l3)
    return full[:USER_N], full[USER_N:]
